# Initial kernel scaffold; baseline (speedup 1.0000x reference)
import jax
import jax.numpy as jnp
from jax.experimental import pallas as pl


def kernel(x_maps, x_rois):
    raise NotImplementedError("write your pallas kernel here")


# speedup vs baseline: 5.8212x; 5.8212x over previous
import functools

import jax
import jax.numpy as jnp
from jax import lax
from jax.experimental import pallas as pl
from jax.experimental.pallas import tpu as pltpu
from jax.experimental.pallas import tpu_sc as plsc

POOL = 7
WIN = 32
CC = 64
LANES = 16
NW = 32
NG = CC // LANES


def _roi_pool_sc(fmap, params):
    nroi = params.shape[0]
    nb, hh, ww, c_total = fmap.shape
    nchunk = c_total // CC
    rois_per_w = nroi // NW
    r_per_b = nroi // nb

    mesh = plsc.VectorSubcoreMesh(core_axis_name="c", subcore_axis_name="s")

    @functools.partial(
        pl.kernel,
        out_type=jax.ShapeDtypeStruct((nroi, POOL, POOL, c_total), jnp.float32),
        mesh=mesh,
        compiler_params=pltpu.CompilerParams(use_tc_tiling_on_sc=False),
        scratch_types=[
            pltpu.VMEM((WIN, WIN, CC), jnp.float32),
            pltpu.VMEM((POOL, POOL, CC), jnp.float32),
            pltpu.VMEM((nroi, 2 * LANES), jnp.int32),
        ],
    )
    def k(fmap_hbm, params_hbm, out_hbm, win_v, out_v, par_v):
        wid = lax.axis_index("s") * 2 + lax.axis_index("c")
        pltpu.sync_copy(params_hbm, par_v)

        def roi_body(i, carry):
            roi = wid * rois_per_w + i
            b = roi // r_per_b
            vec0 = par_v[roi, pl.ds(0, LANES)]
            vec1 = par_v[roi, pl.ds(LANES, LANES)]
            yy = vec0[0]
            xx = vec0[1]

            def cc_body(ci, carry2):
                pltpu.sync_copy(
                    fmap_hbm.at[b, pl.ds(yy, WIN), pl.ds(xx, WIN),
                                pl.ds(ci * CC, CC)],
                    win_v)

                for py in range(POOL):
                    r0 = vec0[2 + py]
                    rn = vec0[9 + py]
                    for px in range(POOL):
                        c0 = vec1[px]
                        cn = vec1[POOL + px]

                        def row_body(r, accs):
                            def col_body(c, accs2):
                                return tuple(
                                    jnp.maximum(
                                        accs2[g],
                                        win_v[r, c, pl.ds(g * LANES, LANES)])
                                    for g in range(NG))
                            return lax.fori_loop(c0, c0 + cn, col_body, accs)

                        neg = jnp.full((LANES,), -jnp.inf, jnp.float32)
                        accs = lax.fori_loop(r0, r0 + rn, row_body, (neg,) * NG)
                        for g in range(NG):
                            out_v[py, px, pl.ds(g * LANES, LANES)] = accs[g]

                pltpu.sync_copy(out_v,
                                out_hbm.at[roi, :, :, pl.ds(ci * CC, CC)])
                return carry2

            lax.fori_loop(0, nchunk, cc_body, 0)
            return carry

        lax.fori_loop(0, rois_per_w, roi_body, 0)

    return k(fmap, params)


def kernel(x_maps, x_rois):
    B, H, W, C = x_maps.shape
    R = x_rois.shape[1]
    y = x_rois[..., 0].astype(jnp.int32)
    x = x_rois[..., 1].astype(jnp.int32)
    h = x_rois[..., 2].astype(jnp.int32)
    w = x_rois[..., 3].astype(jnp.int32)

    p = jnp.arange(POOL, dtype=jnp.int32)
    y0 = (p * h[..., None]) // POOL
    y1 = ((p + 1) * h[..., None]) // POOL
    ys = jnp.maximum(y1 - y0, 1)
    x0 = (p * w[..., None]) // POOL
    x1 = ((p + 1) * w[..., None]) // POOL
    xs = jnp.maximum(x1 - x0, 1)

    nroi = B * R
    zero2 = jnp.zeros((B, R, 2), jnp.int32)
    params = jnp.concatenate(
        [y[..., None], x[..., None], y0, ys, x0, xs, zero2],
        axis=-1).reshape(nroi, 2 * LANES).astype(jnp.int32)

    out = _roi_pool_sc(x_maps, params)
    return out.reshape(B, R, POOL, POOL, C)
